# Initial kernel scaffold; baseline (speedup 1.0000x reference)
import jax
import jax.numpy as jnp
from jax.experimental import pallas as pl


def kernel(input_ids, token_table, pos_table, ln_gamma, ln_beta):
    raise NotImplementedError("write your pallas kernel here")


# speedup vs baseline: 1.1977x; 1.1977x over previous
import functools

import jax
import jax.numpy as jnp
from jax import lax
from jax.experimental import pallas as pl
from jax.experimental.pallas import tpu as pltpu
from jax.experimental.pallas import tpu_sc as plsc

NUM_CORES = 2
NUM_SUBCORES = 16
NUM_WORKERS = NUM_CORES * NUM_SUBCORES
GATHER_CHUNK = 128

TC_BLOCK = 512


def _sc_gather(table, idx2d, tokens, hidden):
    rows_per_worker = tokens // NUM_WORKERS
    chunks = rows_per_worker // GATHER_CHUNK
    mesh = plsc.VectorSubcoreMesh(core_axis_name="c", subcore_axis_name="s")

    @functools.partial(
        pl.kernel,
        out_type=jax.ShapeDtypeStruct((tokens, hidden), jnp.float32),
        mesh=mesh,
        scratch_types=[
            pltpu.VMEM((chunks, GATHER_CHUNK), jnp.int32),
            pltpu.VMEM((rows_per_worker, hidden), jnp.float32),
            pltpu.SemaphoreType.DMA,
        ],
    )
    def gather_kernel(table_hbm, idx_hbm, out_hbm, idx_v, rows_v, sem):
        wid = lax.axis_index("s") * NUM_CORES + lax.axis_index("c")
        base = wid * rows_per_worker
        pltpu.sync_copy(idx_hbm.at[pl.ds(wid * chunks, chunks)], idx_v)
        copies = []
        for j in range(chunks):
            copies.append(
                pltpu.async_copy(
                    table_hbm.at[idx_v.at[j]],
                    rows_v.at[pl.ds(j * GATHER_CHUNK, GATHER_CHUNK)],
                    sem,
                )
            )
        for cp in copies:
            cp.wait()
        pltpu.sync_copy(rows_v, out_hbm.at[pl.ds(base, rows_per_worker)])

    return gather_kernel(table, idx2d)


def _tc_add_ln(gathered, table, gamma, beta, tokens, seq, hidden):
    def body(g_ref, p_ref, gm_ref, bt_ref, o_ref):
        e = g_ref[...] + p_ref[...]
        m = jnp.mean(e, axis=1, keepdims=True)
        d = e - m
        v = jnp.mean(d * d, axis=1, keepdims=True)
        o_ref[...] = d * lax.rsqrt(v + 1e-12) * gm_ref[...] + bt_ref[...]

    pos_blocks = seq // TC_BLOCK
    return pl.pallas_call(
        body,
        grid=(tokens // TC_BLOCK,),
        in_specs=[
            pl.BlockSpec((TC_BLOCK, hidden), lambda i: (i, 0)),
            pl.BlockSpec((TC_BLOCK, hidden), lambda i: (i % pos_blocks, 0)),
            pl.BlockSpec((1, hidden), lambda i: (0, 0)),
            pl.BlockSpec((1, hidden), lambda i: (0, 0)),
        ],
        out_specs=pl.BlockSpec((TC_BLOCK, hidden), lambda i: (i, 0)),
        out_shape=jax.ShapeDtypeStruct((tokens, hidden), jnp.float32),
    )(gathered, table, gamma.reshape(1, hidden), beta.reshape(1, hidden))


@jax.jit
def _impl(input_ids, token_table, pos_table, ln_gamma, ln_beta):
    batch, seq = input_ids.shape
    hidden = token_table.shape[1]
    tokens = batch * seq
    idx2d = input_ids.astype(jnp.int32).reshape(tokens // GATHER_CHUNK, GATHER_CHUNK)
    gathered = _sc_gather(token_table, idx2d, tokens, hidden)
    out = _tc_add_ln(gathered, token_table, ln_gamma, ln_beta, tokens, seq, hidden)
    return out.reshape(batch, seq, hidden)


def kernel(input_ids, token_table, pos_table, ln_gamma, ln_beta):
    return _impl(input_ids, token_table, pos_table, ln_gamma, ln_beta)


# speedup vs baseline: 1.2227x; 1.0209x over previous
import functools

import jax
import jax.numpy as jnp
from jax import lax
from jax.experimental import pallas as pl
from jax.experimental.pallas import tpu as pltpu
from jax.experimental.pallas import tpu_sc as plsc

NUM_CORES = 2
NUM_SUBCORES = 16
NUM_WORKERS = NUM_CORES * NUM_SUBCORES
GATHER_CHUNK = 128

TC_BLOCK = 512


def _sc_gather(table, ids, tokens, hidden):
    batch, seq = ids.shape
    rows_per_worker = tokens // NUM_WORKERS
    chunks = rows_per_worker // GATHER_CHUNK
    workers_per_row = seq // rows_per_worker
    mesh = plsc.VectorSubcoreMesh(core_axis_name="c", subcore_axis_name="s")

    @functools.partial(
        pl.kernel,
        out_type=jax.ShapeDtypeStruct((tokens, hidden), jnp.float32),
        mesh=mesh,
        scratch_types=[
            pltpu.VMEM((rows_per_worker,), jnp.int32),
            pltpu.VMEM((rows_per_worker, hidden), jnp.float32),
            pltpu.SemaphoreType.DMA,
        ],
    )
    def gather_kernel(table_hbm, idx_hbm, out_hbm, idx_v, rows_v, sem):
        wid = lax.axis_index("s") * NUM_CORES + lax.axis_index("c")
        base = wid * rows_per_worker
        b = wid // workers_per_row
        col = (wid % workers_per_row) * rows_per_worker
        pltpu.sync_copy(idx_hbm.at[b, pl.ds(col, rows_per_worker)], idx_v)
        copies = []
        for j in range(chunks):
            copies.append(
                pltpu.async_copy(
                    table_hbm.at[idx_v.at[pl.ds(j * GATHER_CHUNK, GATHER_CHUNK)]],
                    rows_v.at[pl.ds(j * GATHER_CHUNK, GATHER_CHUNK)],
                    sem,
                )
            )
        for cp in copies:
            cp.wait()
        pltpu.sync_copy(rows_v, out_hbm.at[pl.ds(base, rows_per_worker)])

    return gather_kernel(table, ids)


def _tc_add_ln(gathered, table, gamma, beta, tokens, seq, hidden):
    def body(g_ref, p_ref, gm_ref, bt_ref, o_ref):
        e = g_ref[...] + p_ref[...]
        m = jnp.mean(e, axis=1, keepdims=True)
        d = e - m
        v = jnp.mean(d * d, axis=1, keepdims=True)
        o_ref[...] = d * lax.rsqrt(v + 1e-12) * gm_ref[...] + bt_ref[...]

    pos_blocks = seq // TC_BLOCK
    batch = tokens // seq
    return pl.pallas_call(
        body,
        grid=(pos_blocks, batch),
        in_specs=[
            pl.BlockSpec((TC_BLOCK, hidden), lambda j, b: (b * pos_blocks + j, 0)),
            pl.BlockSpec((TC_BLOCK, hidden), lambda j, b: (j, 0)),
            pl.BlockSpec((1, hidden), lambda j, b: (0, 0)),
            pl.BlockSpec((1, hidden), lambda j, b: (0, 0)),
        ],
        out_specs=pl.BlockSpec((TC_BLOCK, hidden), lambda j, b: (b * pos_blocks + j, 0)),
        out_shape=jax.ShapeDtypeStruct((tokens, hidden), jnp.float32),
    )(gathered, table, gamma.reshape(1, hidden), beta.reshape(1, hidden))


@jax.jit
def _impl(input_ids, token_table, pos_table, ln_gamma, ln_beta):
    batch, seq = input_ids.shape
    hidden = token_table.shape[1]
    tokens = batch * seq
    gathered = _sc_gather(token_table, input_ids.astype(jnp.int32), tokens, hidden)
    out = _tc_add_ln(gathered, token_table, ln_gamma, ln_beta, tokens, seq, hidden)
    return out.reshape(batch, seq, hidden)


def kernel(input_ids, token_table, pos_table, ln_gamma, ln_beta):
    return _impl(input_ids, token_table, pos_table, ln_gamma, ln_beta)


# speedup vs baseline: 1.4667x; 1.1996x over previous
import functools

import jax
import jax.numpy as jnp
from jax import lax
from jax.experimental import pallas as pl
from jax.experimental.pallas import tpu as pltpu
from jax.experimental.pallas import tpu_sc as plsc

NUM_CORES = 2
NUM_SUBCORES = 16
NUM_WORKERS = NUM_CORES * NUM_SUBCORES
GATHER_CHUNK = 128

TC_BLOCK = 2048


def _sc_gather(table, ids, tokens, hidden):
    batch, seq = ids.shape
    rows_per_worker = tokens // NUM_WORKERS
    chunks = rows_per_worker // GATHER_CHUNK
    workers_per_row = seq // rows_per_worker
    mesh = plsc.VectorSubcoreMesh(core_axis_name="c", subcore_axis_name="s")

    @functools.partial(
        pl.kernel,
        out_type=jax.ShapeDtypeStruct((tokens, hidden), jnp.float32),
        mesh=mesh,
        scratch_types=[
            pltpu.VMEM((rows_per_worker,), jnp.int32),
            pltpu.VMEM((rows_per_worker, hidden), jnp.float32),
            pltpu.SemaphoreType.DMA,
        ],
    )
    def gather_kernel(table_hbm, idx_hbm, out_hbm, idx_v, rows_v, sem):
        wid = lax.axis_index("s") * NUM_CORES + lax.axis_index("c")
        base = wid * rows_per_worker
        b = wid // workers_per_row
        col = (wid % workers_per_row) * rows_per_worker
        pltpu.sync_copy(idx_hbm.at[b, pl.ds(col, rows_per_worker)], idx_v)
        copies = []
        for j in range(chunks):
            copies.append(
                pltpu.async_copy(
                    table_hbm.at[idx_v.at[pl.ds(j * GATHER_CHUNK, GATHER_CHUNK)]],
                    rows_v.at[pl.ds(j * GATHER_CHUNK, GATHER_CHUNK)],
                    sem,
                )
            )
        for cp in copies:
            cp.wait()
        pltpu.sync_copy(rows_v, out_hbm.at[pl.ds(base, rows_per_worker)])

    return gather_kernel(table, ids)


def _tc_add_ln(gathered, table, gamma, beta, tokens, seq, hidden):
    def body(g_ref, p_ref, gm_ref, bt_ref, o_ref):
        e = g_ref[...] + p_ref[...]
        m = jnp.mean(e, axis=1, keepdims=True)
        s2 = jnp.mean(e * e, axis=1, keepdims=True)
        k = lax.rsqrt(s2 - m * m + 1e-12)
        o_ref[...] = (e - m) * k * gm_ref[...] + bt_ref[...]

    pos_blocks = seq // TC_BLOCK
    batch = tokens // seq
    return pl.pallas_call(
        body,
        grid=(pos_blocks, batch),
        in_specs=[
            pl.BlockSpec((TC_BLOCK, hidden), lambda j, b: (b * pos_blocks + j, 0)),
            pl.BlockSpec((TC_BLOCK, hidden), lambda j, b: (j, 0)),
            pl.BlockSpec((1, hidden), lambda j, b: (0, 0)),
            pl.BlockSpec((1, hidden), lambda j, b: (0, 0)),
        ],
        out_specs=pl.BlockSpec((TC_BLOCK, hidden), lambda j, b: (b * pos_blocks + j, 0)),
        out_shape=jax.ShapeDtypeStruct((tokens, hidden), jnp.float32),
    )(gathered, table, gamma.reshape(1, hidden), beta.reshape(1, hidden))


@jax.jit
def _impl(input_ids, token_table, pos_table, ln_gamma, ln_beta):
    batch, seq = input_ids.shape
    hidden = token_table.shape[1]
    tokens = batch * seq
    gathered = _sc_gather(token_table, input_ids.astype(jnp.int32), tokens, hidden)
    out = _tc_add_ln(gathered, token_table, ln_gamma, ln_beta, tokens, seq, hidden)
    return out.reshape(batch, seq, hidden)


def kernel(input_ids, token_table, pos_table, ln_gamma, ln_beta):
    return _impl(input_ids, token_table, pos_table, ln_gamma, ln_beta)
